# Initial kernel scaffold; baseline (speedup 1.0000x reference)
import jax
import jax.numpy as jnp
from jax.experimental import pallas as pl


def kernel(preds_boxes, preds_classes, anchors, y_boxes, y_classes):
    raise NotImplementedError("write your pallas kernel here")


# speedup vs baseline: 2.7411x; 2.7411x over previous
import functools

import jax
import jax.numpy as jnp
from jax.experimental import pallas as pl
from jax.experimental.pallas import tpu as pltpu

_TA = 1024


def _iou_tile(yb_ref, an_ref, o_pad):
    acx = an_ref[0, 0:1, :]
    acy = an_ref[0, 1:2, :]
    aw = an_ref[0, 2:3, :]
    ah = an_ref[0, 3:4, :]
    ax1 = acx - aw / 2
    ay1 = acy - ah / 2
    ax2 = acx + aw / 2
    ay2 = acy + ah / 2
    gx1 = yb_ref[0, :, 0:1]
    gy1 = yb_ref[0, :, 1:2]
    gx2 = yb_ref[0, :, 2:3]
    gy2 = yb_ref[0, :, 3:4]
    ltx = jnp.maximum(gx1, ax1)
    lty = jnp.maximum(gy1, ay1)
    rbx = jnp.minimum(gx2, ax2)
    rby = jnp.minimum(gy2, ay2)
    wx = jnp.maximum(rbx - ltx, 0.0)
    wy = jnp.maximum(rby - lty, 0.0)
    inter = wx * wy
    ag = (gx2 - gx1) * (gy2 - gy1)
    aa = (ax2 - ax1) * (ay2 - ay1)
    union = ag + aa - inter
    return inter / (union + 1e-8)


def _pass1_body(yb_ref, an_ref, best_ref, rmax_ref, *, o_pad, ta):
    t = pl.program_id(1)
    iou = _iou_tile(yb_ref, an_ref, o_pad)
    tmax = jnp.max(iou, axis=1, keepdims=True)
    lane = jax.lax.broadcasted_iota(jnp.int32, (o_pad, ta), 1) + t * ta
    tidx = jnp.min(jnp.where(iou == tmax, lane, jnp.int32(2**30)),
                   axis=1, keepdims=True)

    @pl.when(t == 0)
    def _():
        rmax_ref[0, :, :] = tmax
        best_ref[0, :, :] = tidx

    @pl.when(t != 0)
    def _():
        cur = rmax_ref[0, :, :]
        curi = best_ref[0, :, :]
        upd = tmax > cur
        rmax_ref[0, :, :] = jnp.where(upd, tmax, cur)
        best_ref[0, :, :] = jnp.where(upd, tidx, curi)


def _pass2_body(yb_ref, yc_ref, best_ref, an_ref, pb_ref, pc_ref, out_ref,
                *, o_pad, ta, n_real_o, n_real_a):
    t = pl.program_id(1)
    iou = _iou_tile(yb_ref, an_ref, o_pad)

    row = jax.lax.broadcasted_iota(jnp.int32, (o_pad, 1), 0)
    rvalid = row < n_real_o
    lane = jax.lax.broadcasted_iota(jnp.int32, (o_pad, ta), 1) + t * ta
    lane1 = jax.lax.broadcasted_iota(jnp.int32, (1, ta), 1) + t * ta
    avalid = lane1 < n_real_a

    best_col = best_ref[0, :, :]
    pos = (iou >= 0.5) | ((lane == best_col) & rvalid)
    pos_f = pos.astype(jnp.float32)

    gx1 = yb_ref[0, :, 0:1]
    gy1 = yb_ref[0, :, 1:2]
    gx2 = yb_ref[0, :, 2:3]
    gy2 = yb_ref[0, :, 3:4]
    gcx = (gx1 + gx2) / 2
    gcy = (gy1 + gy2) / 2
    gw = jnp.where(rvalid, gx2 - gx1, 1.0)
    gh = jnp.where(rvalid, gy2 - gy1, 1.0)
    rv = rvalid.astype(jnp.float32)
    lw = jnp.log(gw)
    lh = jnp.log(gh)
    yc = yc_ref[0, :, 0:1]
    t1 = (yc == 0.0).astype(jnp.float32)
    t2 = (yc == 1.0).astype(jnp.float32)
    u = jnp.concatenate(
        [rv, gcx * rv, gcy * rv, gcx * gcx * rv, gcy * gcy * rv,
         lw, lh, lw * lw + lh * lh, t1, t2,
         jnp.zeros((o_pad, 6), jnp.float32)], axis=1)

    n = jax.lax.dot_general(u, pos_f, (((0,), (0,)), ((), ())),
                            preferred_element_type=jnp.float32,
                            precision=jax.lax.Precision.HIGHEST)

    p_cnt = n[0:1, :]
    s1 = n[8:9, :]
    s2 = n[9:10, :]

    acx = an_ref[0, 0:1, :]
    acy = an_ref[0, 1:2, :]
    awp = an_ref[0, 2:3, :] + 1e-8
    ahp = an_ref[0, 3:4, :] + 1e-8
    rx = 1.0 / awp
    ry = 1.0 / ahp
    pbx = pb_ref[0, 0:1, :]
    pby = pb_ref[0, 1:2, :]
    pbw = pb_ref[0, 2:3, :]
    pbh = pb_ref[0, 3:4, :]
    qx = pbx + acx * rx
    qy = pby + acy * ry
    sw = pbw + jnp.log(awp)
    sh = pbh + jnp.log(ahp)

    box_per_a = ((qx * qx + qy * qy + sw * sw + sh * sh) * p_cnt
                 + (-2.0 * qx * rx) * n[1:2, :]
                 + (-2.0 * qy * ry) * n[2:3, :]
                 + (rx * rx) * n[3:4, :]
                 + (ry * ry) * n[4:5, :]
                 + (-2.0 * sw) * n[5:6, :]
                 + (-2.0 * sh) * n[6:7, :]
                 + n[7:8, :])
    s_box = jnp.sum(box_per_a)

    pc0 = pc_ref[0, 0:1, :]
    pc1 = pc_ref[0, 1:2, :]
    pc2 = pc_ref[0, 2:3, :]

    def focal(x):
        p = jax.nn.sigmoid(x)
        stab = jnp.log1p(jnp.exp(-jnp.abs(x)))
        mx = jnp.maximum(x, 0.0)
        ce1 = mx - x + stab
        ce0 = mx + stab
        lt1 = 0.8 * ce1 * jnp.sqrt(1.0 - p)
        lt0 = 0.2 * ce0 * jnp.sqrt(p)
        return lt1, lt0

    lt1_0, lt0_0 = focal(pc0)
    lt1_1, lt0_1 = focal(pc1)
    lt1_2, lt0_2 = focal(pc2)

    s_pcls = jnp.sum(p_cnt * lt0_0
                     + s1 * lt1_1 + (p_cnt - s1) * lt0_1
                     + s2 * lt1_2 + (p_cnt - s2) * lt0_2)

    m_a = jnp.max(iou, axis=0, keepdims=True)
    neg_f = ((m_a < 0.4) & avalid).astype(jnp.float32)
    s_ncls = jnp.sum(neg_f * (lt1_0 + lt0_1 + lt0_2))

    is0 = (pc0 >= pc1) & (pc0 >= pc2)
    is1 = jnp.logical_not(is0) & (pc1 >= pc2)
    is2 = jnp.logical_not(is0 | is1)
    s_pm = jnp.sum(s1 * is1.astype(jnp.float32) + s2 * is2.astype(jnp.float32))
    s_nm = jnp.sum(neg_f * is0.astype(jnp.float32))

    s_np = jnp.sum(p_cnt)
    s_nn = jnp.sum(neg_f)

    li = jax.lax.broadcasted_iota(jnp.int32, (8, 128), 1)
    ri = jax.lax.broadcasted_iota(jnp.int32, (8, 128), 0)
    row0 = ri == 0

    def slot(k):
        return ((li == k) & row0).astype(jnp.float32)

    contrib = (s_np * slot(0) + s_nn * slot(1) + s_box * slot(2)
               + s_pcls * slot(3) + s_ncls * slot(4)
               + s_pm * slot(5) + s_nm * slot(6))

    first = (pl.program_id(0) == 0) & (t == 0)

    @pl.when(first)
    def _():
        out_ref[:, :] = contrib

    @pl.when(jnp.logical_not(first))
    def _():
        out_ref[:, :] = out_ref[:, :] + contrib


def kernel(preds_boxes, preds_classes, anchors, y_boxes, y_classes):
    B, A, _ = preds_boxes.shape
    O = y_boxes.shape[1]
    a_pad = ((A + _TA - 1) // _TA) * _TA
    nt = a_pad // _TA
    o_pad = ((O + 7) // 8) * 8

    an_t = jnp.pad(jnp.transpose(anchors, (0, 2, 1)),
                   ((0, 0), (0, 0), (0, a_pad - A)))
    pb_t = jnp.pad(jnp.transpose(preds_boxes, (0, 2, 1)),
                   ((0, 0), (0, 0), (0, a_pad - A)))
    pc_t = jnp.pad(jnp.transpose(preds_classes, (0, 2, 1)),
                   ((0, 0), (0, 0), (0, a_pad - A)))
    yb = jnp.pad(y_boxes, ((0, 0), (0, o_pad - O), (0, 0)))
    yc = jnp.pad(y_classes.astype(jnp.float32)[..., None],
                 ((0, 0), (0, o_pad - O), (0, 0)), constant_values=-1.0)

    gt_spec = pl.BlockSpec((1, o_pad, 4), lambda b, t: (b, 0, 0))
    gt1_spec = pl.BlockSpec((1, o_pad, 1), lambda b, t: (b, 0, 0))
    a4_spec = pl.BlockSpec((1, 4, _TA), lambda b, t: (b, 0, t))
    a3_spec = pl.BlockSpec((1, 3, _TA), lambda b, t: (b, 0, t))

    best, _rmax = pl.pallas_call(
        functools.partial(_pass1_body, o_pad=o_pad, ta=_TA),
        grid=(B, nt),
        in_specs=[gt_spec, a4_spec],
        out_specs=[gt1_spec, gt1_spec],
        out_shape=[jax.ShapeDtypeStruct((B, o_pad, 1), jnp.int32),
                   jax.ShapeDtypeStruct((B, o_pad, 1), jnp.float32)],
        compiler_params=pltpu.CompilerParams(
            dimension_semantics=("arbitrary", "arbitrary")),
    )(yb, an_t)

    acc = pl.pallas_call(
        functools.partial(_pass2_body, o_pad=o_pad, ta=_TA,
                          n_real_o=O, n_real_a=A),
        grid=(B, nt),
        in_specs=[gt_spec, gt1_spec, gt1_spec, a4_spec, a4_spec, a3_spec],
        out_specs=pl.BlockSpec((8, 128), lambda b, t: (0, 0)),
        out_shape=jax.ShapeDtypeStruct((8, 128), jnp.float32),
        compiler_params=pltpu.CompilerParams(
            dimension_semantics=("arbitrary", "arbitrary")),
    )(yb, yc, best, an_t, pb_t, pc_t)

    n_p = acc[0, 0]
    n_n = acc[0, 1]
    s_box = acc[0, 2]
    s_pcls = acc[0, 3]
    s_ncls = acc[0, 4]
    s_pm = acc[0, 5]
    s_nm = acc[0, 6]
    boxes_loss = 0.01 * s_box / (4.0 * n_p)
    classes_loss = (s_pcls + s_ncls) / (3.0 * (n_p + n_n))
    total = boxes_loss + classes_loss
    f1 = (s_pm + s_nm) / (n_p + n_n)
    return (total, boxes_loss, classes_loss, f1)


# speedup vs baseline: 4.0586x; 1.4807x over previous
import functools

import jax
import jax.numpy as jnp
from jax.experimental import pallas as pl
from jax.experimental.pallas import tpu as pltpu

_TA = 2048


def _body(yb_ref, yc_ref, an_ref, pb_ref, pc_ref, out_ref,
          rmax_ref, cbest_ref, *, o_pad, ta, nt, n_real_o, n_real_a):
    t = pl.program_id(1)

    acx = an_ref[0, 0:1, :]
    acy = an_ref[0, 1:2, :]
    aw = an_ref[0, 2:3, :]
    ah = an_ref[0, 3:4, :]
    ax1 = acx - aw / 2
    ay1 = acy - ah / 2
    ax2 = acx + aw / 2
    ay2 = acy + ah / 2
    gx1 = yb_ref[0, :, 0:1]
    gy1 = yb_ref[0, :, 1:2]
    gx2 = yb_ref[0, :, 2:3]
    gy2 = yb_ref[0, :, 3:4]
    ltx = jnp.maximum(gx1, ax1)
    lty = jnp.maximum(gy1, ay1)
    rbx = jnp.minimum(gx2, ax2)
    rby = jnp.minimum(gy2, ay2)
    wx = jnp.maximum(rbx - ltx, 0.0)
    wy = jnp.maximum(rby - lty, 0.0)
    inter = wx * wy
    ag = (gx2 - gx1) * (gy2 - gy1)
    aa = (ax2 - ax1) * (ay2 - ay1)
    union = ag + aa - inter
    iou = inter / (union + 1e-8)

    row = jax.lax.broadcasted_iota(jnp.int32, (o_pad, 1), 0)
    rvalid = row < n_real_o
    rv = rvalid.astype(jnp.float32)

    awp = aw + 1e-8
    ahp = ah + 1e-8
    rx = 1.0 / awp
    ry = 1.0 / ahp
    pbx = pb_ref[0, 0:1, :]
    pby = pb_ref[0, 1:2, :]
    pbw = pb_ref[0, 2:3, :]
    pbh = pb_ref[0, 3:4, :]
    qx = pbx + acx * rx
    qy = pby + acy * ry
    sw = pbw + jnp.log(awp)
    sh = pbh + jnp.log(ahp)
    c0 = qx * qx + qy * qy + sw * sw + sh * sh
    c1 = -2.0 * qx * rx
    c2 = -2.0 * qy * ry
    c3 = rx * rx
    c4 = ry * ry
    c5 = -2.0 * sw
    c6 = -2.0 * sh

    pc0 = pc_ref[0, 0:1, :]
    pc1 = pc_ref[0, 1:2, :]
    pc2 = pc_ref[0, 2:3, :]

    def focal(x):
        p = jax.nn.sigmoid(x)
        stab = jnp.log1p(jnp.exp(-jnp.abs(x)))
        mx = jnp.maximum(x, 0.0)
        ce1 = mx - x + stab
        ce0 = mx + stab
        return 0.8 * ce1 * jnp.sqrt(1.0 - p), 0.2 * ce0 * jnp.sqrt(p)

    lt1_0, lt0_0 = focal(pc0)
    lt1_1, lt0_1 = focal(pc1)
    lt1_2, lt0_2 = focal(pc2)
    lt0sum = lt0_0 + lt0_1 + lt0_2
    d1 = lt1_1 - lt0_1
    d2 = lt1_2 - lt0_2

    is0 = (pc0 >= pc1) & (pc0 >= pc2)
    is1 = jnp.logical_not(is0) & (pc1 >= pc2)
    is2 = jnp.logical_not(is0 | is1)
    e1 = is1.astype(jnp.float32)
    e2 = is2.astype(jnp.float32)

    ones = jnp.ones((1, ta), jnp.float32)

    gcx = (gx1 + gx2) / 2
    gcy = (gy1 + gy2) / 2
    gw = jnp.where(rvalid, gx2 - gx1, 1.0)
    gh = jnp.where(rvalid, gy2 - gy1, 1.0)
    lw = jnp.log(gw)
    lh = jnp.log(gh)
    yc = yc_ref[0, :, 0:1]
    t1 = (yc == 0.0).astype(jnp.float32)
    t2 = (yc == 1.0).astype(jnp.float32)
    u = jnp.concatenate(
        [rv, gcx * rv, gcy * rv, gcx * gcx * rv, gcy * gcy * rv,
         lw, lh, lw * lw + lh * lh, t1, t2, rv, t1, t2, t1, t2,
         jnp.zeros((o_pad, 1), jnp.float32)], axis=1)

    pos_f = (iou >= 0.5).astype(jnp.float32)
    n = jax.lax.dot_general(u, pos_f, (((0,), (0,)), ((), ())),
                            preferred_element_type=jnp.float32,
                            precision=jax.lax.Precision.HIGHEST)
    p_cnt = n[0:1, :]
    s1 = n[8:9, :]
    s2 = n[9:10, :]

    box_per_a = (c0 * p_cnt + c1 * n[1:2, :] + c2 * n[2:3, :]
                 + c3 * n[3:4, :] + c4 * n[4:5, :]
                 + c5 * n[5:6, :] + c6 * n[6:7, :] + n[7:8, :])
    s_box = jnp.sum(box_per_a)
    s_pcls = jnp.sum(p_cnt * lt0sum + s1 * d1 + s2 * d2)
    s_pm = jnp.sum(s1 * e1 + s2 * e2)
    s_np = jnp.sum(p_cnt)

    lane1 = jax.lax.broadcasted_iota(jnp.int32, (1, ta), 1) + t * ta
    avalid = lane1 < n_real_a
    m_a = jnp.max(iou, axis=0, keepdims=True)
    neg_f = ((m_a < 0.4) & avalid).astype(jnp.float32)
    s_ncls = jnp.sum(neg_f * (lt1_0 + lt0_1 + lt0_2))
    s_nm = jnp.sum(neg_f * is0.astype(jnp.float32))
    s_nn = jnp.sum(neg_f)

    tmax = jnp.max(iou, axis=1, keepdims=True)
    lane = jax.lax.broadcasted_iota(jnp.int32, (o_pad, ta), 1)
    tidx = jnp.min(jnp.where(iou == tmax, lane, jnp.int32(2**30)),
                   axis=1, keepdims=True)
    oh = (lane == tidx).astype(jnp.float32)
    cfeat = jnp.concatenate(
        [c0, c1, c2, c3, c4, c5, c6, ones,
         jnp.zeros((2, ta), jnp.float32),
         lt0sum, d1, d2, e1, e2,
         jnp.zeros((1, ta), jnp.float32)], axis=0)
    g = jax.lax.dot_general(oh, cfeat, (((1,), (1,)), ((), ())),
                            preferred_element_type=jnp.float32,
                            precision=jax.lax.Precision.HIGHEST)

    is_first = t == 0
    cur_max = jnp.where(is_first, jnp.float32(-1.0), rmax_ref[:, :])
    upd = tmax > cur_max
    new_max = jnp.where(upd, tmax, cur_max)
    rmax_ref[:, :] = new_max
    cur_cb = jnp.where(is_first, jnp.zeros_like(g), cbest_ref[:, :])
    new_cb = jnp.where(upd, g, cur_cb)
    cbest_ref[:, :] = new_cb

    li16 = jax.lax.broadcasted_iota(jnp.int32, (o_pad, 16), 1)
    box_m = (li16 < 8).astype(jnp.float32)
    cls_m = ((li16 >= 10) & (li16 < 13)).astype(jnp.float32)
    pm_m = ((li16 >= 13) & (li16 < 15)).astype(jnp.float32)
    last = t == nt - 1
    fix_f = jnp.where(last, (rvalid & (new_max < 0.5)).astype(jnp.float32),
                      jnp.zeros((o_pad, 1), jnp.float32))
    prod = u * new_cb * fix_f
    f_box = jnp.sum(prod * box_m)
    f_pcls = jnp.sum(prod * cls_m)
    f_pm = jnp.sum(prod * pm_m)
    f_np = jnp.sum(fix_f)

    li = jax.lax.broadcasted_iota(jnp.int32, (8, 128), 1)
    ri = jax.lax.broadcasted_iota(jnp.int32, (8, 128), 0)
    row0 = ri == 0
    vals = (s_np + f_np, s_nn, s_box + f_box, s_pcls + f_pcls,
            s_ncls, s_pm + f_pm, s_nm)
    contrib = jnp.zeros((8, 128), jnp.float32)
    for k, v in enumerate(vals):
        contrib = contrib + v * ((li == k) & row0).astype(jnp.float32)

    first = (pl.program_id(0) == 0) & (t == 0)

    @pl.when(first)
    def _():
        out_ref[:, :] = contrib

    @pl.when(jnp.logical_not(first))
    def _():
        out_ref[:, :] = out_ref[:, :] + contrib


def kernel(preds_boxes, preds_classes, anchors, y_boxes, y_classes):
    B, A, _ = preds_boxes.shape
    O = y_boxes.shape[1]
    a_pad = ((A + _TA - 1) // _TA) * _TA
    nt = a_pad // _TA
    o_pad = ((O + 7) // 8) * 8

    an_t = jnp.pad(jnp.transpose(anchors, (0, 2, 1)),
                   ((0, 0), (0, 0), (0, a_pad - A)))
    pb_t = jnp.pad(jnp.transpose(preds_boxes, (0, 2, 1)),
                   ((0, 0), (0, 0), (0, a_pad - A)))
    pc_t = jnp.pad(jnp.transpose(preds_classes, (0, 2, 1)),
                   ((0, 0), (0, 0), (0, a_pad - A)))
    yb = jnp.pad(y_boxes, ((0, 0), (0, o_pad - O), (0, 0)))
    yc = jnp.pad(y_classes.astype(jnp.float32)[..., None],
                 ((0, 0), (0, o_pad - O), (0, 0)), constant_values=-1.0)

    acc = pl.pallas_call(
        functools.partial(_body, o_pad=o_pad, ta=_TA, nt=nt,
                          n_real_o=O, n_real_a=A),
        grid=(B, nt),
        in_specs=[pl.BlockSpec((1, o_pad, 4), lambda b, t: (b, 0, 0)),
                  pl.BlockSpec((1, o_pad, 1), lambda b, t: (b, 0, 0)),
                  pl.BlockSpec((1, 4, _TA), lambda b, t: (b, 0, t)),
                  pl.BlockSpec((1, 4, _TA), lambda b, t: (b, 0, t)),
                  pl.BlockSpec((1, 3, _TA), lambda b, t: (b, 0, t))],
        out_specs=pl.BlockSpec((8, 128), lambda b, t: (0, 0)),
        out_shape=jax.ShapeDtypeStruct((8, 128), jnp.float32),
        scratch_shapes=[pltpu.VMEM((o_pad, 1), jnp.float32),
                        pltpu.VMEM((o_pad, 16), jnp.float32)],
        compiler_params=pltpu.CompilerParams(
            dimension_semantics=("arbitrary", "arbitrary")),
    )(yb, yc, an_t, pb_t, pc_t)

    n_p = acc[0, 0]
    n_n = acc[0, 1]
    s_box = acc[0, 2]
    s_pcls = acc[0, 3]
    s_ncls = acc[0, 4]
    s_pm = acc[0, 5]
    s_nm = acc[0, 6]
    boxes_loss = 0.01 * s_box / (4.0 * n_p)
    classes_loss = (s_pcls + s_ncls) / (3.0 * (n_p + n_n))
    total = boxes_loss + classes_loss
    f1 = (s_pm + s_nm) / (n_p + n_n)
    return (total, boxes_loss, classes_loss, f1)
